# baseline (speedup 1.0000x reference)
import functools

import jax
import jax.numpy as jnp
from jax import lax
from jax.experimental import pallas as pl
from jax.experimental.pallas import tpu as pltpu
from jax.experimental.pallas import tpu_sc as plsc


def _greedy_sc(xT):
    V, U, B = xT.shape
    info = plsc.get_sparse_core_info()
    NC, NS, L = info.num_cores, info.num_subcores, info.num_lanes
    NW = NC * NS
    SPW = B // NW
    NG = SPW // L
    mesh = plsc.VectorSubcoreMesh(core_axis_name="c", subcore_axis_name="s")

    @functools.partial(
        pl.kernel,
        out_type=(
            jax.ShapeDtypeStruct((B,), jnp.float32),
            jax.ShapeDtypeStruct((V, B), jnp.int32),
        ),
        mesh=mesh,
        scratch_types=(
            [pltpu.VMEM((U, SPW), jnp.float32) for _ in range(2)]
            + [
                pltpu.VMEM((U, SPW), jnp.float32),
                pltpu.VMEM((V, SPW), jnp.int32),
                pltpu.VMEM((SPW,), jnp.float32),
            ]
            + [pltpu.SemaphoreType.DMA for _ in range(2)]
        ),
        compiler_params=pltpu.CompilerParams(
            needs_layout_passes=False, use_tc_tiling_on_sc=False
        ),
    )
    def greedy(xT_hbm, size_hbm, seq_hbm, st0, st1, mbuf, seqacc, sizebuf,
               sem0, sem1):
        stages = (st0, st1)
        sems = (sem0, sem1)
        wid = lax.axis_index("s") * NC + lax.axis_index("c")
        base = wid * SPW
        iota = lax.iota(jnp.int32, L)
        zero_v = jnp.zeros((L,), jnp.float32)
        neg2_v = jnp.full((L,), -2.0, dtype=jnp.float32)

        @pl.loop(0, U)
        def zero_m(r):
            for g in range(NG):
                mbuf[r, pl.ds(g * L, L)] = zero_v

        def start(slot, t):
            pltpu.async_copy(
                xT_hbm.at[t, :, pl.ds(base, SPW)], stages[slot], sems[slot]
            )

        def wait(slot):
            pltpu.make_async_copy(
                xT_hbm.at[0, :, pl.ds(base, SPW)], stages[slot], sems[slot]
            ).wait()

        def step(slot, t, sizes):
            stg = stages[slot]
            t_vec = jnp.full((L,), t, dtype=jnp.int32)
            new_sizes = []
            for g in range(NG):
                c = g * L
                bv = stg[0, pl.ds(c, L)]
                bi = jnp.zeros((L,), jnp.int32)
                for u in range(1, U):
                    wv = stg[u, pl.ds(c, L)] + mbuf[u, pl.ds(c, L)]
                    tk = wv > bv
                    bv = jnp.where(tk, wv, bv)
                    bi = jnp.where(tk, jnp.int32(u), bi)
                new_sizes.append(sizes[g] - bv)
                plsc.store_scatter(
                    mbuf, [bi, iota + c], neg2_v, mask=bi != 0
                )
                plsc.store_scatter(seqacc, [t_vec, iota + c], bi)
            return tuple(new_sizes)

        start(0, 0)
        start(1, 1)

        def pair(i, sizes):
            tt = 2 * i
            wait(0)
            sizes = step(0, tt, sizes)

            @pl.when(tt + 2 < V)
            def _pf0():
                start(0, tt + 2)

            wait(1)
            sizes = step(1, tt + 1, sizes)

            @pl.when(tt + 3 < V)
            def _pf1():
                start(1, tt + 3)

            return sizes

        sizes = lax.fori_loop(
            0, V // 2, pair, tuple(zero_v for _ in range(NG))
        )
        for g in range(NG):
            sizebuf[pl.ds(g * L, L)] = sizes[g]
        pltpu.sync_copy(seqacc, seq_hbm.at[:, pl.ds(base, SPW)])
        pltpu.sync_copy(sizebuf, size_hbm.at[pl.ds(base, SPW)])

    return greedy(xT)


def kernel(x, u_size, v_size):
    del u_size, v_size
    B = x.shape[0]
    H = B // 2
    outs = []
    for h in range(2):
        xh = jnp.transpose(x[h * H : (h + 1) * H], (1, 2, 0))
        outs.append(_greedy_sc(xh))
    neg_size = jnp.concatenate([o[0] for o in outs])
    seqsT = jnp.concatenate([o[1] for o in outs], axis=1)
    return neg_size, jnp.transpose(seqsT, (1, 0))


# speedup vs baseline: 1.5067x; 1.5067x over previous
import functools

import jax
import jax.numpy as jnp
from jax import lax
from jax.experimental import pallas as pl
from jax.experimental.pallas import tpu as pltpu
from jax.experimental.pallas import tpu_sc as plsc


def _greedy_sc(xT):
    V, U, B = xT.shape
    info = plsc.get_sparse_core_info()
    NC, NS, L = info.num_cores, info.num_subcores, info.num_lanes
    NW = NC * NS
    SPW = B // NW
    NG = SPW // L
    mesh = plsc.VectorSubcoreMesh(core_axis_name="c", subcore_axis_name="s")

    @functools.partial(
        pl.kernel,
        out_type=(
            jax.ShapeDtypeStruct((B,), jnp.float32),
            jax.ShapeDtypeStruct((V, B), jnp.int32),
        ),
        mesh=mesh,
        scratch_types=(
            [pltpu.VMEM((U, SPW), jnp.float32) for _ in range(2)]
            + [
                pltpu.VMEM((U, SPW), jnp.float32),
                pltpu.VMEM((V, SPW), jnp.int32),
                pltpu.VMEM((SPW,), jnp.float32),
            ]
            + [pltpu.SemaphoreType.DMA for _ in range(2)]
        ),
        compiler_params=pltpu.CompilerParams(
            needs_layout_passes=False, use_tc_tiling_on_sc=False
        ),
    )
    def greedy(xT_hbm, size_hbm, seq_hbm, st0, st1, mbuf, seqacc, sizebuf,
               sem0, sem1):
        stages = (st0, st1)
        sems = (sem0, sem1)
        wid = lax.axis_index("s") * NC + lax.axis_index("c")
        base = wid * SPW
        iota = lax.iota(jnp.int32, L)
        zero_v = jnp.zeros((L,), jnp.float32)
        neg2_v = jnp.full((L,), -2.0, dtype=jnp.float32)

        @pl.loop(0, U)
        def zero_m(r):
            for g in range(NG):
                mbuf[r, pl.ds(g * L, L)] = zero_v

        def start(slot, t):
            pltpu.async_copy(
                xT_hbm.at[t, :, pl.ds(base, SPW)], stages[slot], sems[slot]
            )

        def wait(slot):
            pltpu.make_async_copy(
                xT_hbm.at[0, :, pl.ds(base, SPW)], stages[slot], sems[slot]
            ).wait()

        def step(slot, t, sizes):
            stg = stages[slot]
            t_vec = jnp.full((L,), t, dtype=jnp.int32)
            new_sizes = []
            for g in range(NG):
                c = g * L
                bv = stg[0, pl.ds(c, L)]
                bi = jnp.zeros((L,), jnp.int32)
                for u in range(1, U):
                    wv = stg[u, pl.ds(c, L)] + mbuf[u, pl.ds(c, L)]
                    tk = wv > bv
                    bv = jnp.where(tk, wv, bv)
                    bi = jnp.where(tk, jnp.int32(u), bi)
                new_sizes.append(sizes[g] - bv)
                plsc.store_scatter(
                    mbuf, [bi, iota + c], neg2_v, mask=bi != 0
                )
                plsc.store_scatter(seqacc, [t_vec, iota + c], bi)
            return tuple(new_sizes)

        start(0, 0)
        start(1, 1)

        def pair(i, sizes):
            tt = 2 * i
            wait(0)
            sizes = step(0, tt, sizes)

            @pl.when(tt + 2 < V)
            def _pf0():
                start(0, tt + 2)

            wait(1)
            sizes = step(1, tt + 1, sizes)

            @pl.when(tt + 3 < V)
            def _pf1():
                start(1, tt + 3)

            return sizes

        sizes = lax.fori_loop(
            0, V // 2, pair, tuple(zero_v for _ in range(NG))
        )
        for g in range(NG):
            sizebuf[pl.ds(g * L, L)] = sizes[g]
        pltpu.sync_copy(seqacc, seq_hbm.at[:, pl.ds(base, SPW)])
        pltpu.sync_copy(sizebuf, size_hbm.at[pl.ds(base, SPW)])

    return greedy(xT)


def kernel(x, u_size, v_size):
    del u_size, v_size
    xT = jnp.transpose(x, (1, 2, 0))
    neg_size, seqsT = _greedy_sc(xT)
    return neg_size, jnp.transpose(seqsT, (1, 0))


# speedup vs baseline: 1.5520x; 1.0301x over previous
import functools

import jax
import jax.numpy as jnp
from jax import lax
from jax.experimental import pallas as pl
from jax.experimental.pallas import tpu as pltpu
from jax.experimental.pallas import tpu_sc as plsc


def _greedy_sc(xT):
    V, U, B = xT.shape
    info = plsc.get_sparse_core_info()
    NC, NS, L = info.num_cores, info.num_subcores, info.num_lanes
    NW = NC * NS
    SPW = B // NW
    NG = SPW // L
    mesh = plsc.VectorSubcoreMesh(core_axis_name="c", subcore_axis_name="s")

    @functools.partial(
        pl.kernel,
        out_type=(
            jax.ShapeDtypeStruct((B,), jnp.float32),
            jax.ShapeDtypeStruct((V, B), jnp.int32),
        ),
        mesh=mesh,
        scratch_types=(
            [pltpu.VMEM((U, SPW), jnp.float32) for _ in range(2)]
            + [
                pltpu.VMEM((U, SPW), jnp.float32),
                pltpu.VMEM((V, SPW), jnp.int32),
                pltpu.VMEM((SPW,), jnp.float32),
            ]
            + [pltpu.SemaphoreType.DMA for _ in range(2)]
        ),
        compiler_params=pltpu.CompilerParams(
            needs_layout_passes=False, use_tc_tiling_on_sc=False
        ),
    )
    def greedy(xT_hbm, size_hbm, seq_hbm, st0, st1, mbuf, seqacc, sizebuf,
               sem0, sem1):
        stages = (st0, st1)
        sems = (sem0, sem1)
        wid = lax.axis_index("s") * NC + lax.axis_index("c")
        base = wid * SPW
        iota = lax.iota(jnp.int32, L)
        zero_v = jnp.zeros((L,), jnp.float32)
        neg2_v = jnp.full((L,), -2.0, dtype=jnp.float32)

        @pl.loop(0, U)
        def zero_m(r):
            for g in range(NG):
                mbuf[r, pl.ds(g * L, L)] = zero_v

        def start(slot, t):
            pltpu.async_copy(
                xT_hbm.at[t, :, pl.ds(base, SPW)], stages[slot], sems[slot]
            )

        def wait(slot):
            pltpu.make_async_copy(
                xT_hbm.at[0, :, pl.ds(base, SPW)], stages[slot], sems[slot]
            ).wait()

        def step(slot, t, sizes):
            stg = stages[slot]
            t_vec = jnp.full((L,), t, dtype=jnp.int32)
            new_sizes = []
            for g in range(NG):
                c = g * L
                bv0 = stg[0, pl.ds(c, L)]
                bi0 = jnp.zeros((L,), jnp.int32)

                @pl.loop(
                    1, U - ((U - 1) % 4), step=4, init_carry=(bv0, bi0)
                )
                def scan_u(u0, carry):
                    bv, bi = carry
                    for du in range(4):
                        u = u0 + du
                        wv = stg[u, pl.ds(c, L)] + mbuf[u, pl.ds(c, L)]
                        tk = wv > bv
                        bv = jnp.where(tk, wv, bv)
                        bi = jnp.where(tk, u.astype(jnp.int32), bi)
                    return bv, bi

                bv, bi = scan_u
                for u in range(U - ((U - 1) % 4), U):
                    wv = stg[u, pl.ds(c, L)] + mbuf[u, pl.ds(c, L)]
                    tk = wv > bv
                    bv = jnp.where(tk, wv, bv)
                    bi = jnp.where(tk, jnp.int32(u), bi)
                new_sizes.append(sizes[g] - bv)
                plsc.store_scatter(
                    mbuf, [bi, iota + c], neg2_v, mask=bi != 0
                )
                plsc.store_scatter(seqacc, [t_vec, iota + c], bi)
            return tuple(new_sizes)

        start(0, 0)
        start(1, 1)

        def pair(i, sizes):
            tt = 2 * i
            wait(0)
            sizes = step(0, tt, sizes)

            @pl.when(tt + 2 < V)
            def _pf0():
                start(0, tt + 2)

            wait(1)
            sizes = step(1, tt + 1, sizes)

            @pl.when(tt + 3 < V)
            def _pf1():
                start(1, tt + 3)

            return sizes

        sizes = lax.fori_loop(
            0, V // 2, pair, tuple(zero_v for _ in range(NG))
        )
        for g in range(NG):
            sizebuf[pl.ds(g * L, L)] = sizes[g]
        pltpu.sync_copy(seqacc, seq_hbm.at[:, pl.ds(base, SPW)])
        pltpu.sync_copy(sizebuf, size_hbm.at[pl.ds(base, SPW)])

    return greedy(xT)


def kernel(x, u_size, v_size):
    del u_size, v_size
    xT = jnp.transpose(x, (1, 2, 0))
    neg_size, seqsT = _greedy_sc(xT)
    return neg_size, jnp.transpose(seqsT, (1, 0))


# speedup vs baseline: 1.5584x; 1.0041x over previous
import functools

import jax
import jax.numpy as jnp
from jax import lax
from jax.experimental import pallas as pl
from jax.experimental.pallas import tpu as pltpu
from jax.experimental.pallas import tpu_sc as plsc


def _greedy_sc(xT):
    V, U, B = xT.shape
    info = plsc.get_sparse_core_info()
    NC, NS, L = info.num_cores, info.num_subcores, info.num_lanes
    NW = NC * NS
    SPW = B // NW
    NG = SPW // L
    mesh = plsc.VectorSubcoreMesh(core_axis_name="c", subcore_axis_name="s")

    @functools.partial(
        pl.kernel,
        out_type=(
            jax.ShapeDtypeStruct((B,), jnp.float32),
            jax.ShapeDtypeStruct((V, B), jnp.int32),
        ),
        mesh=mesh,
        scratch_types=(
            [pltpu.VMEM((U, SPW), jnp.float32) for _ in range(2)]
            + [
                pltpu.VMEM((U, SPW), jnp.float32),
                pltpu.VMEM((V, SPW), jnp.int32),
                pltpu.VMEM((SPW,), jnp.float32),
            ]
            + [pltpu.SemaphoreType.DMA for _ in range(2)]
        ),
        compiler_params=pltpu.CompilerParams(
            needs_layout_passes=False, use_tc_tiling_on_sc=False
        ),
    )
    def greedy(xT_hbm, size_hbm, seq_hbm, st0, st1, mbuf, seqacc, sizebuf,
               sem0, sem1):
        stages = (st0, st1)
        sems = (sem0, sem1)
        wid = lax.axis_index("s") * NC + lax.axis_index("c")
        base = wid * SPW
        iota = lax.iota(jnp.int32, L)
        zero_v = jnp.zeros((L,), jnp.float32)
        neg2_v = jnp.full((L,), -2.0, dtype=jnp.float32)

        @pl.loop(0, U)
        def zero_m(r):
            for g in range(NG):
                mbuf[r, pl.ds(g * L, L)] = zero_v

        def start(slot, t):
            pltpu.async_copy(
                xT_hbm.at[t, :, pl.ds(base, SPW)], stages[slot], sems[slot]
            )

        def wait(slot):
            pltpu.make_async_copy(
                xT_hbm.at[0, :, pl.ds(base, SPW)], stages[slot], sems[slot]
            ).wait()

        def step(slot, t, sizes):
            stg = stages[slot]
            t_vec = jnp.full((L,), t, dtype=jnp.int32)
            new_sizes = []
            for g in range(NG):
                c = g * L
                bv0 = stg[0, pl.ds(c, L)]
                bi0 = jnp.zeros((L,), jnp.int32)

                @pl.loop(
                    1, U - ((U - 1) % 8), step=8, init_carry=(bv0, bi0)
                )
                def scan_u(u0, carry):
                    bv, bi = carry
                    for du in range(8):
                        u = u0 + du
                        wv = stg[u, pl.ds(c, L)] + mbuf[u, pl.ds(c, L)]
                        tk = wv > bv
                        bv = jnp.where(tk, wv, bv)
                        bi = jnp.where(tk, u.astype(jnp.int32), bi)
                    return bv, bi

                bv, bi = scan_u
                for u in range(U - ((U - 1) % 8), U):
                    wv = stg[u, pl.ds(c, L)] + mbuf[u, pl.ds(c, L)]
                    tk = wv > bv
                    bv = jnp.where(tk, wv, bv)
                    bi = jnp.where(tk, jnp.int32(u), bi)
                new_sizes.append(sizes[g] - bv)
                plsc.store_scatter(
                    mbuf, [bi, iota + c], neg2_v, mask=bi != 0
                )
                plsc.store_scatter(seqacc, [t_vec, iota + c], bi)
            return tuple(new_sizes)

        start(0, 0)
        start(1, 1)

        def pair(i, sizes):
            tt = 2 * i
            wait(0)
            sizes = step(0, tt, sizes)

            @pl.when(tt + 2 < V)
            def _pf0():
                start(0, tt + 2)

            wait(1)
            sizes = step(1, tt + 1, sizes)

            @pl.when(tt + 3 < V)
            def _pf1():
                start(1, tt + 3)

            return sizes

        sizes = lax.fori_loop(
            0, V // 2, pair, tuple(zero_v for _ in range(NG))
        )
        for g in range(NG):
            sizebuf[pl.ds(g * L, L)] = sizes[g]
        pltpu.sync_copy(seqacc, seq_hbm.at[:, pl.ds(base, SPW)])
        pltpu.sync_copy(sizebuf, size_hbm.at[pl.ds(base, SPW)])

    return greedy(xT)


def kernel(x, u_size, v_size):
    del u_size, v_size
    xT = jnp.transpose(x, (1, 2, 0))
    neg_size, seqsT = _greedy_sc(xT)
    return neg_size, jnp.transpose(seqsT, (1, 0))
